# baseline (speedup 1.0000x reference)
import functools

import jax
import jax.numpy as jnp
from jax import lax
from jax.experimental import pallas as pl
from jax.experimental.pallas import tpu as pltpu
from jax.experimental.pallas import tpu_sc as plsc

NUM_ITEM = 1000000
DIM = 32
BATCH = 16384

_info = plsc.get_sparse_core_info()
_NC, _NS, _L = _info.num_cores, _info.num_subcores, _info.num_lanes
_NW = _NC * _NS
_BLK = 512
_BLKP = _BLK + 5
_CPW = 61
_NFULL = _CPW * _NW
_TAILSTART = _NFULL * _BLK
_TAIL64 = NUM_ITEM - 64
_PSLOTS = 130 * 128
_OUTPAD = 128

_mesh = plsc.VectorSubcoreMesh(core_axis_name="c", subcore_axis_name="s")
_params = pltpu.CompilerParams(needs_layout_passes=False)


@functools.partial(
    pl.kernel,
    mesh=_mesh,
    out_type=jax.ShapeDtypeStruct((BATCH + _OUTPAD,), jnp.float32),
    scratch_types=[
        pltpu.VMEM((DIM, _BLKP), jnp.float32),
        pltpu.VMEM((DIM, _BLKP), jnp.float32),
        pltpu.VMEM((DIM, 64), jnp.float32),
        pltpu.VMEM((BATCH,), jnp.int32),
        pltpu.VMEM((BATCH,), jnp.int32),
        pltpu.VMEM((130, 128), jnp.int32),
        pltpu.VMEM((_PSLOTS,), jnp.float32),
        pltpu.VMEM((DIM, _L), jnp.float32),
        pltpu.SemaphoreType.DMA,
        pltpu.SemaphoreType.DMA,
        pltpu.SemaphoreType.DMA,
    ],
    compiler_params=_params,
)
def _sc_stream_score(idx_hbm, userb_hbm, tt_hbm, out_hbm,
                     tbuf0, tbuf1, ttail, idxall, proc,
                     comppos, valbuf, u_v, sin0, sin1, sout):
    wid = lax.axis_index("s") * _NC + lax.axis_index("c")
    iota = lax.iota(jnp.int32, _L)
    tbufs = (tbuf0, tbuf1)
    sins = (sin0, sin1)
    lo = wid * (_CPW * _BLK)
    hi = jnp.where(wid == _NW - 1, NUM_ITEM, lo + _CPW * _BLK)

    pltpu.sync_copy(idx_hbm, idxall)
    pltpu.sync_copy(userb_hbm, u_v)

    def bin_body(v, off):
        rawv = idxall[pl.ds(v * _L, _L)]
        m = (rawv >= lo) & (rawv < hi)
        mi = m.astype(jnp.int32)
        pc = plsc.cumsum(mi)
        slot = off + pc - 1
        plsc.store_scatter(idxall, [slot], rawv, mask=m)
        plsc.store_scatter(
            comppos,
            [lax.shift_right_logical(slot, 7), slot & 127],
            v * _L + iota, mask=m)
        return off + jnp.sum(mi)

    m_total = 512

    padval = jnp.full((_L,), BATCH, jnp.int32)
    for k in range(_OUTPAD // _L):
        slot = m_total + k * _L + iota
        plsc.store_scatter(
            comppos,
            [lax.shift_right_logical(slot, 7), slot & 127],
            padval, mask=slot < _PSLOTS)

    nv = lax.shift_right_logical(m_total + (_L - 1), 4)

    def process(tb, start, width):
        def scan_body(v, cnt):
            rawv = idxall[pl.ds(v * _L, _L)]
            m = (rawv >= start) & (rawv < start + width)
            mi = m.astype(jnp.int32)
            pc = plsc.cumsum(mi)
            slot = cnt + pc - 1
            packed = (v * _L + iota) * _BLK + (rawv - start)
            plsc.store_scatter(proc, [slot], packed, mask=m)
            return cnt + jnp.sum(mi)

        cnt = lax.fori_loop(0, nv, scan_body, 0)
        nd = lax.shift_right_logical(cnt + (_L - 1), 4)

        def dot_body(t, carry):
            packed = proc[pl.ds(t * _L, _L)]
            itemv = packed & (_BLK - 1)
            slotv = lax.shift_right_logical(packed, 9)
            acc = jnp.zeros((_L,), jnp.float32)
            for j in range(DIM):
                vals = plsc.load_gather(
                    tb, [jnp.full((_L,), j, jnp.int32), itemv])
                acc = acc + vals * u_v[j]
            wm = (t * _L + iota) < cnt
            plsc.store_scatter(valbuf, [slotv], acc, mask=wm)
            return carry

        lax.fori_loop(0, nd, dot_body, 0)

    def cin(s, blk):
        return pltpu.make_async_copy(
            tt_hbm.at[:, pl.ds(blk * _BLK, _BLK)],
            tbufs[s].at[:, pl.ds(0, _BLK)], sins[s])

    blk0 = wid * _CPW
    cin(0, blk0).start()

    def pair(k, carry):
        for s in (0, 1):
            i = 2 * k + s
            blk = blk0 + i

            @pl.when(i < _CPW - 1)
            def _():
                cin(1 - s, blk + 1).start()

            cin(s, blk).wait()
        return carry

    lax.fori_loop(0, (_CPW - 1) // 2, pair, 0)
    lastblk = blk0 + _CPW - 1
    cin(0, lastblk).wait()
    process(tbufs[0], lastblk * _BLK, _BLK)

    @pl.when(wid == _NW - 1)
    def _():
        cin(0, _NFULL).start()
        cin(0, _NFULL).wait()
        process(tbufs[0], _TAILSTART, _BLK)
        pltpu.sync_copy(tt_hbm.at[:, pl.ds(_TAIL64, 64)], ttail)
        process(ttail, _TAIL64, 64)

    nt = lax.shift_right_logical(m_total + 127, 7)

    def scat_body(t, carry):
        pltpu.async_copy(
            valbuf.at[pl.ds(t * 128, 128)],
            out_hbm.at[comppos.at[t]],
            sout).wait()
        return carry

    lax.fori_loop(0, nt, scat_body, 0)


def kernel(item_idx, user_emb, item_emb):
    idx = item_idx.astype(jnp.int32)
    userb = jnp.broadcast_to(user_emb.reshape(DIM, 1), (DIM, _L))
    out = _sc_stream_score(idx, userb, item_emb.T)
    return out[:BATCH].reshape(1, BATCH)


# speedup vs baseline: 1.0636x; 1.0636x over previous
import functools

import jax
import jax.numpy as jnp
from jax import lax
from jax.experimental import pallas as pl
from jax.experimental.pallas import tpu as pltpu
from jax.experimental.pallas import tpu_sc as plsc

NUM_ITEM = 1000000
DIM = 32
BATCH = 16384

_info = plsc.get_sparse_core_info()
_NC, _NS, _L = _info.num_cores, _info.num_subcores, _info.num_lanes
_NW = _NC * _NS
_BLK = 512
_BLKP = _BLK
_CPW = 61
_NFULL = _CPW * _NW
_TAILSTART = _NFULL * _BLK
_TAIL64 = NUM_ITEM - 64
_PSLOTS = 130 * 128
_OUTPAD = 128

_mesh = plsc.VectorSubcoreMesh(core_axis_name="c", subcore_axis_name="s")
_params = pltpu.CompilerParams(needs_layout_passes=False)


@functools.partial(
    pl.kernel,
    mesh=_mesh,
    out_type=jax.ShapeDtypeStruct((BATCH + _OUTPAD,), jnp.float32),
    scratch_types=[
        pltpu.VMEM((DIM, _BLKP), jnp.float32),
        pltpu.VMEM((DIM, _BLKP), jnp.float32),
        pltpu.VMEM((DIM, 64), jnp.float32),
        pltpu.VMEM((BATCH,), jnp.int32),
        pltpu.VMEM((BATCH,), jnp.int32),
        pltpu.VMEM((130, 128), jnp.int32),
        pltpu.VMEM((_PSLOTS,), jnp.float32),
        pltpu.VMEM((DIM, _L), jnp.float32),
        pltpu.SemaphoreType.DMA,
        pltpu.SemaphoreType.DMA,
        pltpu.SemaphoreType.DMA,
    ],
    compiler_params=_params,
)
def _sc_stream_score(idx_hbm, userb_hbm, tt_hbm, out_hbm,
                     tbuf0, tbuf1, ttail, idxall, proc,
                     comppos, valbuf, u_v, sin0, sin1, sout):
    wid = lax.axis_index("s") * _NC + lax.axis_index("c")
    iota = lax.iota(jnp.int32, _L)
    tbufs = (tbuf0, tbuf1)
    sins = (sin0, sin1)
    lo = wid * (_CPW * _BLK)
    hi = jnp.where(wid == _NW - 1, NUM_ITEM, lo + _CPW * _BLK)

    pltpu.sync_copy(idx_hbm, idxall)
    pltpu.sync_copy(userb_hbm, u_v)

    def bin_body(v, off):
        rawv = idxall[pl.ds(v * _L, _L)]
        m = (rawv >= lo) & (rawv < hi)
        mi = m.astype(jnp.int32)
        pc = plsc.cumsum(mi)
        slot = off + pc - 1
        plsc.store_scatter(idxall, [slot], rawv, mask=m)
        plsc.store_scatter(
            comppos,
            [lax.shift_right_logical(slot, 7), slot & 127],
            v * _L + iota, mask=m)
        return off + jnp.sum(mi)

    m_total = lax.fori_loop(0, BATCH // _L, bin_body, 0)

    padval = jnp.full((_L,), BATCH, jnp.int32)
    for k in range(_OUTPAD // _L):
        slot = m_total + k * _L + iota
        plsc.store_scatter(
            comppos,
            [lax.shift_right_logical(slot, 7), slot & 127],
            padval, mask=slot < _PSLOTS)

    nv = lax.shift_right_logical(m_total + (_L - 1), 4)

    def process(tb, start, width):
        def scan_body(v, cnt):
            rawv = idxall[pl.ds(v * _L, _L)]
            m = (rawv >= start) & (rawv < start + width)
            mi = m.astype(jnp.int32)
            pc = plsc.cumsum(mi)
            slot = cnt + pc - 1
            packed = (v * _L + iota) * _BLK + (rawv - start)
            plsc.store_scatter(proc, [slot], packed, mask=m)
            return cnt + jnp.sum(mi)

        cnt = lax.fori_loop(0, nv, scan_body, 0)
        nd = lax.shift_right_logical(cnt + (_L - 1), 4)

        def dot_body(t, carry):
            packed = proc[pl.ds(t * _L, _L)]
            itemv = packed & (_BLK - 1)
            slotv = lax.shift_right_logical(packed, 9)
            acc = jnp.zeros((_L,), jnp.float32)
            for j in range(DIM):
                vals = plsc.load_gather(
                    tb, [jnp.full((_L,), j, jnp.int32), itemv])
                acc = acc + vals * u_v[j]
            wm = (t * _L + iota) < cnt
            plsc.store_scatter(valbuf, [slotv], acc, mask=wm)
            return carry

        lax.fori_loop(0, nd, dot_body, 0)

    def cin(s, blk):
        return pltpu.make_async_copy(
            tt_hbm.at[:, pl.ds(blk * _BLK, _BLK)],
            tbufs[s], sins[s])

    blk0 = wid * _CPW
    cin(0, blk0).start()

    def pair(k, carry):
        for s in (0, 1):
            i = 2 * k + s
            blk = blk0 + i

            @pl.when(i < _CPW - 1)
            def _():
                cin(1 - s, blk + 1).start()

            cin(s, blk).wait()
            process(tbufs[s], blk * _BLK, _BLK)
        return carry

    lax.fori_loop(0, (_CPW - 1) // 2, pair, 0)
    lastblk = blk0 + _CPW - 1
    cin(0, lastblk).wait()
    process(tbufs[0], lastblk * _BLK, _BLK)

    @pl.when(wid == _NW - 1)
    def _():
        cin(0, _NFULL).start()
        cin(0, _NFULL).wait()
        process(tbufs[0], _TAILSTART, _BLK)
        pltpu.sync_copy(tt_hbm.at[:, pl.ds(_TAIL64, 64)], ttail)
        process(ttail, _TAIL64, 64)

    nt = lax.shift_right_logical(m_total + 127, 7)

    def scat_body(t, carry):
        pltpu.async_copy(
            valbuf.at[pl.ds(t * 128, 128)],
            out_hbm.at[comppos.at[t]],
            sout).wait()
        return carry

    lax.fori_loop(0, nt, scat_body, 0)


def kernel(item_idx, user_emb, item_emb):
    idx = item_idx.astype(jnp.int32)
    userb = jnp.broadcast_to(user_emb.reshape(DIM, 1), (DIM, _L))
    out = _sc_stream_score(idx, userb, item_emb.T)
    return out[:BATCH].reshape(1, BATCH)


# speedup vs baseline: 1.0638x; 1.0001x over previous
import functools

import jax
import jax.numpy as jnp
from jax import lax
from jax.experimental import pallas as pl
from jax.experimental.pallas import tpu as pltpu
from jax.experimental.pallas import tpu_sc as plsc

NUM_ITEM = 1000000
DIM = 32
BATCH = 16384

_info = plsc.get_sparse_core_info()
_NC, _NS, _L = _info.num_cores, _info.num_subcores, _info.num_lanes
_NW = _NC * _NS
_BLK = 512
_BLKP = _BLK
_CPW = 61
_NFULL = _CPW * _NW
_TAILSTART = _NFULL * _BLK
_TAIL64 = NUM_ITEM - 64
_PSLOTS = 130 * 128
_OUTPAD = 128

_mesh = plsc.VectorSubcoreMesh(core_axis_name="c", subcore_axis_name="s")
_params = pltpu.CompilerParams(needs_layout_passes=False)


@functools.partial(
    pl.kernel,
    mesh=_mesh,
    out_type=jax.ShapeDtypeStruct((BATCH + _OUTPAD,), jnp.float32),
    scratch_types=[
        pltpu.VMEM((DIM, _BLKP), jnp.float32),
        pltpu.VMEM((DIM, _BLKP), jnp.float32),
        pltpu.VMEM((DIM, 64), jnp.float32),
        pltpu.VMEM((BATCH,), jnp.int32),
        pltpu.VMEM((BATCH,), jnp.int32),
        pltpu.VMEM((130, 128), jnp.int32),
        pltpu.VMEM((_PSLOTS,), jnp.float32),
        pltpu.VMEM((DIM, _L), jnp.float32),
        pltpu.SemaphoreType.DMA,
        pltpu.SemaphoreType.DMA,
        pltpu.SemaphoreType.DMA,
    ],
    compiler_params=_params,
)
def _sc_stream_score(idx_hbm, userb_hbm, tt_hbm, out_hbm,
                     tbuf0, tbuf1, ttail, idxall, proc,
                     comppos, valbuf, u_v, sin0, sin1, sout):
    wid = lax.axis_index("s") * _NC + lax.axis_index("c")
    iota = lax.iota(jnp.int32, _L)
    tbufs = (tbuf0, tbuf1)
    sins = (sin0, sin1)
    lo = wid * (_CPW * _BLK)
    hi = jnp.where(wid == _NW - 1, NUM_ITEM, lo + _CPW * _BLK)

    pltpu.sync_copy(idx_hbm, idxall)
    pltpu.sync_copy(userb_hbm, u_v)

    def bin_body(v, off):
        rawv = idxall[pl.ds(v * _L, _L)]
        m = (rawv >= lo) & (rawv < hi)
        mi = m.astype(jnp.int32)
        pc = plsc.cumsum(mi)
        slot = off + pc - 1
        plsc.store_scatter(idxall, [slot], rawv, mask=m)
        plsc.store_scatter(
            comppos,
            [lax.shift_right_logical(slot, 7), slot & 127],
            v * _L + iota, mask=m)
        return off + jnp.sum(mi)

    m_total = lax.fori_loop(0, BATCH // _L, bin_body, 0)

    padval = jnp.full((_L,), BATCH, jnp.int32)
    for k in range(_OUTPAD // _L):
        slot = m_total + k * _L + iota
        plsc.store_scatter(
            comppos,
            [lax.shift_right_logical(slot, 7), slot & 127],
            padval, mask=slot < _PSLOTS)

    nv = lax.shift_right_logical(m_total + (_L - 1), 4)

    def process(tb, start, width):
        def scan_body(v, cnt):
            rawv = idxall[pl.ds(v * _L, _L)]
            m = (rawv >= start) & (rawv < start + width)
            mi = m.astype(jnp.int32)
            pc = plsc.cumsum(mi)
            slot = cnt + pc - 1
            packed = (v * _L + iota) * _BLK + (rawv - start)
            plsc.store_scatter(proc, [slot], packed, mask=m)
            return cnt + jnp.sum(mi)

        cnt = lax.fori_loop(0, nv, scan_body, 0)
        nd = lax.shift_right_logical(cnt + (_L - 1), 4)

        def dot_body(t, carry):
            packed = proc[pl.ds(t * _L, _L)]
            itemv = packed & (_BLK - 1)
            slotv = lax.shift_right_logical(packed, 9)
            acc = jnp.zeros((_L,), jnp.float32)
            for j in range(DIM):
                vals = plsc.load_gather(
                    tb, [jnp.full((_L,), j, jnp.int32), itemv])
                acc = acc + vals * u_v[j]
            wm = (t * _L + iota) < cnt
            plsc.store_scatter(valbuf, [slotv], acc, mask=wm)
            return carry

        lax.fori_loop(0, nd, dot_body, 0)

    def cin_all(s, blk):
        return [pltpu.make_async_copy(
            tt_hbm.at[pl.ds(tr * 8, 8), pl.ds(blk * _BLK, _BLK)],
            tbufs[s].at[pl.ds(tr * 8, 8), :], sins[s])
            for tr in range(DIM // 8)]

    def cin_start(s, blk):
        for c in cin_all(s, blk):
            c.start()

    def cin_wait(s, blk):
        for c in cin_all(s, blk):
            c.wait()

    blk0 = wid * _CPW
    cin_start(0, blk0)

    def pair(k, carry):
        for s in (0, 1):
            i = 2 * k + s
            blk = blk0 + i

            @pl.when(i < _CPW - 1)
            def _():
                cin_start(1 - s, blk + 1)

            cin_wait(s, blk)
            process(tbufs[s], blk * _BLK, _BLK)
        return carry

    lax.fori_loop(0, (_CPW - 1) // 2, pair, 0)
    lastblk = blk0 + _CPW - 1
    cin_wait(0, lastblk)
    process(tbufs[0], lastblk * _BLK, _BLK)

    @pl.when(wid == _NW - 1)
    def _():
        cin_start(0, _NFULL)
        cin_wait(0, _NFULL)
        process(tbufs[0], _TAILSTART, _BLK)
        pltpu.sync_copy(tt_hbm.at[:, pl.ds(_TAIL64, 64)], ttail)
        process(ttail, _TAIL64, 64)

    nt = lax.shift_right_logical(m_total + 127, 7)

    def scat_body(t, carry):
        pltpu.async_copy(
            valbuf.at[pl.ds(t * 128, 128)],
            out_hbm.at[comppos.at[t]],
            sout).wait()
        return carry

    lax.fori_loop(0, nt, scat_body, 0)


def kernel(item_idx, user_emb, item_emb):
    idx = item_idx.astype(jnp.int32)
    userb = jnp.broadcast_to(user_emb.reshape(DIM, 1), (DIM, _L))
    out = _sc_stream_score(idx, userb, item_emb.T)
    return out[:BATCH].reshape(1, BATCH)


# speedup vs baseline: 1.0987x; 1.0328x over previous
import functools

import jax
import jax.numpy as jnp
from jax import lax
from jax.experimental import pallas as pl
from jax.experimental.pallas import tpu as pltpu
from jax.experimental.pallas import tpu_sc as plsc

NUM_ITEM = 1000000
DIM = 32
BATCH = 16384

_info = plsc.get_sparse_core_info()
_NC, _NS, _L = _info.num_cores, _info.num_subcores, _info.num_lanes
_NW = _NC * _NS
_BLK = 512
_BLKP = _BLK
_CPW = 61
_NFULL = _CPW * _NW
_TAILSTART = _NFULL * _BLK
_TAIL64 = NUM_ITEM - 64
_PSLOTS = 130 * 128
_OUTPAD = 128

_mesh = plsc.VectorSubcoreMesh(core_axis_name="c", subcore_axis_name="s")
_params = pltpu.CompilerParams(needs_layout_passes=False)


@functools.partial(
    pl.kernel,
    mesh=_mesh,
    out_type=jax.ShapeDtypeStruct((BATCH + _OUTPAD,), jnp.float32),
    scratch_types=[
        pltpu.VMEM((DIM, _BLKP), jnp.float32),
        pltpu.VMEM((DIM, _BLKP), jnp.float32),
        pltpu.VMEM((DIM, _BLKP), jnp.float32),
        pltpu.VMEM((DIM, 64), jnp.float32),
        pltpu.VMEM((BATCH,), jnp.int32),
        pltpu.VMEM((BATCH,), jnp.int32),
        pltpu.VMEM((130, 128), jnp.int32),
        pltpu.VMEM((_PSLOTS,), jnp.float32),
        pltpu.VMEM((DIM, _L), jnp.float32),
        pltpu.SemaphoreType.DMA,
        pltpu.SemaphoreType.DMA,
        pltpu.SemaphoreType.DMA,
        pltpu.SemaphoreType.DMA,
    ],
    compiler_params=_params,
)
def _sc_stream_score(idx_hbm, userb_hbm, tt_hbm, out_hbm,
                     tbuf0, tbuf1, tbuf2, ttail, idxall, proc,
                     comppos, valbuf, u_v, sin0, sin1, sin2, sout):
    wid = lax.axis_index("s") * _NC + lax.axis_index("c")
    iota = lax.iota(jnp.int32, _L)
    tbufs = (tbuf0, tbuf1, tbuf2)
    sins = (sin0, sin1, sin2)
    lo = wid * (_CPW * _BLK)
    hi = jnp.where(wid == _NW - 1, NUM_ITEM, lo + _CPW * _BLK)

    pltpu.sync_copy(idx_hbm, idxall)
    pltpu.sync_copy(userb_hbm, u_v)

    def bin_body(v, off):
        rawv = idxall[pl.ds(v * _L, _L)]
        m = (rawv >= lo) & (rawv < hi)
        mi = m.astype(jnp.int32)
        pc = plsc.cumsum(mi)
        slot = off + pc - 1
        plsc.store_scatter(idxall, [slot], rawv, mask=m)
        plsc.store_scatter(
            comppos,
            [lax.shift_right_logical(slot, 7), slot & 127],
            v * _L + iota, mask=m)
        return off + jnp.sum(mi)

    m_total = lax.fori_loop(0, BATCH // _L, bin_body, 0)

    padval = jnp.full((_L,), BATCH, jnp.int32)
    for k in range(_OUTPAD // _L):
        slot = m_total + k * _L + iota
        plsc.store_scatter(
            comppos,
            [lax.shift_right_logical(slot, 7), slot & 127],
            padval, mask=slot < _PSLOTS)

    nv = lax.shift_right_logical(m_total + (_L - 1), 4)

    def process(tb, start, width):
        def scan_body(v, cnt):
            rawv = idxall[pl.ds(v * _L, _L)]
            m = (rawv >= start) & (rawv < start + width)
            mi = m.astype(jnp.int32)
            pc = plsc.cumsum(mi)
            slot = cnt + pc - 1
            packed = (v * _L + iota) * _BLK + (rawv - start)
            plsc.store_scatter(proc, [slot], packed, mask=m)
            return cnt + jnp.sum(mi)

        cnt = lax.fori_loop(0, nv, scan_body, 0)
        nd = lax.shift_right_logical(cnt + (_L - 1), 4)

        def dot_body(t, carry):
            packed = proc[pl.ds(t * _L, _L)]
            itemv = packed & (_BLK - 1)
            slotv = lax.shift_right_logical(packed, 9)
            acc = jnp.zeros((_L,), jnp.float32)
            for j in range(DIM):
                vals = plsc.load_gather(
                    tb, [jnp.full((_L,), j, jnp.int32), itemv])
                acc = acc + vals * u_v[j]
            wm = (t * _L + iota) < cnt
            plsc.store_scatter(valbuf, [slotv], acc, mask=wm)
            return carry

        lax.fori_loop(0, nd, dot_body, 0)

    def cin_all(s, blk):
        return [pltpu.make_async_copy(
            tt_hbm.at[pl.ds(tr * 8, 8), pl.ds(blk * _BLK, _BLK)],
            tbufs[s].at[pl.ds(tr * 8, 8), :], sins[s])
            for tr in range(DIM // 8)]

    def cin_start(s, blk):
        for c in cin_all(s, blk):
            c.start()

    def cin_wait(s, blk):
        for c in cin_all(s, blk):
            c.wait()

    blk0 = wid * _CPW
    cin_start(0, blk0)
    cin_start(1, blk0 + 1)

    def triple(k, carry):
        for s in (0, 1, 2):
            i = 3 * k + s
            blk = blk0 + i

            @pl.when(i < _CPW - 2)
            def _():
                cin_start((s + 2) % 3, blk + 2)

            cin_wait(s, blk)
            process(tbufs[s], blk * _BLK, _BLK)
        return carry

    lax.fori_loop(0, (_CPW - 1) // 3, triple, 0)
    lastblk = blk0 + _CPW - 1
    cin_wait(0, lastblk)
    process(tbufs[0], lastblk * _BLK, _BLK)

    @pl.when(wid == _NW - 1)
    def _():
        cin_start(0, _NFULL)
        cin_wait(0, _NFULL)
        process(tbufs[0], _TAILSTART, _BLK)
        pltpu.sync_copy(tt_hbm.at[:, pl.ds(_TAIL64, 64)], ttail)
        process(ttail, _TAIL64, 64)

    nt = lax.shift_right_logical(m_total + 127, 7)

    def scat_body(t, carry):
        pltpu.async_copy(
            valbuf.at[pl.ds(t * 128, 128)],
            out_hbm.at[comppos.at[t]],
            sout).wait()
        return carry

    lax.fori_loop(0, nt, scat_body, 0)


def kernel(item_idx, user_emb, item_emb):
    idx = item_idx.astype(jnp.int32)
    userb = jnp.broadcast_to(user_emb.reshape(DIM, 1), (DIM, _L))
    out = _sc_stream_score(idx, userb, item_emb.T)
    return out[:BATCH].reshape(1, BATCH)
